# baseline (speedup 1.0000x reference)
import functools
import math

import jax
import jax.numpy as jnp
from jax import lax
from jax.experimental import pallas as pl
from jax.experimental.pallas import tpu as pltpu
from jax.experimental.pallas import tpu_sc as plsc


def _ssp(v):
    return jnp.maximum(v, 0.0) + jnp.log1p(jnp.exp(-jnp.abs(v))) - math.log(2.0)


def _in2f_body(x_ref, w_ref, o_ref):
    o_ref[...] = jnp.dot(x_ref[...], w_ref[...],
                         preferred_element_type=jnp.float32)


def _in2f(x, w):
    n, d = x.shape
    f = w.shape[1]
    blk = 2000
    return pl.pallas_call(
        _in2f_body,
        grid=(n // blk,),
        in_specs=[
            pl.BlockSpec((blk, d), lambda i: (i, 0)),
            pl.BlockSpec((d, f), lambda i: (0, 0)),
        ],
        out_specs=pl.BlockSpec((blk, f), lambda i: (i, 0)),
        out_shape=jax.ShapeDtypeStruct((n, f), jnp.float32),
    )(x, w)


_NC, _NS = 2, 16
_NW = _NC * _NS
_CH = 128
_K = 3
_PH = _CH * _K


def _sc_gather(table, idx, eoff, ecnt):
    n, f = table.shape
    per_w = ecnt // _NW
    nphase = per_w // _PH
    tail = per_w - nphase * _PH
    assert tail % 8 == 0 and tail <= _CH
    mesh = plsc.VectorSubcoreMesh(core_axis_name="c", subcore_axis_name="s")

    @functools.partial(
        pl.kernel,
        out_type=jax.ShapeDtypeStruct((ecnt, f), jnp.float32),
        mesh=mesh,
        scratch_types=[
            pltpu.VMEM((per_w,), jnp.int32),
            pltpu.VMEM((_PH, f), jnp.float32),
            pltpu.VMEM((_PH, f), jnp.float32),
            pltpu.SemaphoreType.DMA,
            pltpu.SemaphoreType.DMA,
        ],
    )
    def k(table_hbm, idx_hbm, out_hbm, idx_v, buf_a, buf_b, sem_a, sem_b):
        wid = lax.axis_index("s") * _NC + lax.axis_index("c")
        base = wid * per_w
        pltpu.sync_copy(idx_hbm.at[pl.ds(eoff + base, per_w)], idx_v)

        def fire(buf, sem, p):
            for j in range(_K):
                pltpu.async_copy(
                    table_hbm.at[idx_v.at[pl.ds(p * _PH + j * _CH, _CH)]],
                    buf.at[pl.ds(j * _CH, _CH)], sem)

        def drain(buf, sem):
            pltpu.make_async_copy(
                table_hbm.at[pl.ds(0, _PH)], buf, sem).wait()

        def flush(buf, p):
            pltpu.sync_copy(buf, out_hbm.at[pl.ds(base + p * _PH, _PH)])

        last = nphase - 1
        fire(buf_a, sem_a, 0)

        def body(i, carry):
            p = 2 * i
            fire(buf_b, sem_b, p + 1)
            drain(buf_a, sem_a)
            flush(buf_a, p)
            fire(buf_a, sem_a, jnp.minimum(p + 2, last))
            drain(buf_b, sem_b)
            flush(buf_b, p + 1)
            return carry

        lax.fori_loop(0, nphase // 2, body, 0, unroll=False)
        drain(buf_a, sem_a)
        if nphase % 2:
            flush(buf_a, last)
        if tail:
            pltpu.async_copy(
                table_hbm.at[idx_v.at[pl.ds(nphase * _PH, tail)]],
                buf_a.at[pl.ds(0, tail)], sem_a).wait()
            pltpu.sync_copy(buf_a.at[pl.ds(0, tail)],
                            out_hbm.at[pl.ds(base + nphase * _PH, tail)])

    return k(table, idx)


def _fused_body(dre_ref, dr_ref, pm_ref, gat_ref,
                fw1_ref, fb1_ref, fw2_ref, fb2_ref,
                f2o_ref, f2ob_ref, ow_ref, ob_ref, o_ref,
                *, blk_n, nbh):
    h3 = lax.dot_general(dre_ref[...], fw1_ref[...], (((0,), (0,)), ((), ())),
                         preferred_element_type=jnp.float32)
    h3 = _ssp(h3 + fb1_ref[...])
    w3 = lax.dot_general(h3, fw2_ref[...], (((2,), (0,)), ((), ())),
                         preferred_element_type=jnp.float32) + fb2_ref[...]
    dr = dr_ref[...]
    cut_t = 0.5 * (jnp.cos(dr * (math.pi / 5.0)) + 1.0)
    cut_t = cut_t * (dr < 5.0).astype(jnp.float32) * pm_ref[...]
    z3 = gat_ref[...] * w3 * cut_t[:, :, None]
    za = z3.sum(axis=0)
    ya = _ssp(jnp.dot(za, f2o_ref[...],
                      preferred_element_type=jnp.float32) + f2ob_ref[...])
    o_ref[...] = jnp.dot(ya, ow_ref[...],
                         preferred_element_type=jnp.float32) + ob_ref[...]


def _fused(dre_t, dr_t, pm_t, gat3, weights, noff, ncnt, nbh, d, f, g):
    fw1, fb1, fw2, fb2, f2o, f2ob, ow, ob = weights
    blk_n = 256
    b0 = noff // blk_n
    grid = (ncnt // blk_n,)
    body = functools.partial(_fused_body, blk_n=blk_n, nbh=nbh)
    return pl.pallas_call(
        body,
        grid=grid,
        in_specs=[
            pl.BlockSpec((g, nbh, blk_n), lambda i: (0, 0, i + b0)),
            pl.BlockSpec((nbh, blk_n), lambda i: (0, i + b0)),
            pl.BlockSpec((nbh, blk_n), lambda i: (0, i + b0)),
            pl.BlockSpec((nbh, blk_n, f), lambda i: (0, i, 0)),
            pl.BlockSpec((g, f), lambda i: (0, 0)),
            pl.BlockSpec((1, f), lambda i: (0, 0)),
            pl.BlockSpec((f, f), lambda i: (0, 0)),
            pl.BlockSpec((1, f), lambda i: (0, 0)),
            pl.BlockSpec((f, d), lambda i: (0, 0)),
            pl.BlockSpec((1, d), lambda i: (0, 0)),
            pl.BlockSpec((d, d), lambda i: (0, 0)),
            pl.BlockSpec((1, d), lambda i: (0, 0)),
        ],
        out_specs=pl.BlockSpec((blk_n, d), lambda i: (i, 0)),
        out_shape=jax.ShapeDtypeStruct((ncnt, d), jnp.float32),
    )(dre_t, dr_t, pm_t, gat3, fw1, fb1, fw2, fb2, f2o, f2ob, ow, ob)


_NCHUNK = 8


def kernel(x, dR, neighbors, pairwise_mask, dR_expanded,
           fW1, fb1, fW2, fb2, in2f_W, f2out_W, f2out_b, out_W, out_b):
    n, nbh = neighbors.shape
    d = x.shape[1]
    f = in2f_W.shape[1]
    g = dR_expanded.shape[2]

    y = _in2f(x, in2f_W)
    np_ = 10240 if n == 10000 else ((n + 639) // 640) * 640
    pad = np_ - n
    nc = np_ // _NCHUNK
    ec = nc * nbh
    pad_idx = jnp.broadcast_to(((jnp.arange(pad) * 977) % n)[None, :],
                               (nbh, pad)).astype(jnp.int32)
    nt = jnp.concatenate([neighbors.T.astype(jnp.int32), pad_idx], axis=1)
    idx = jnp.transpose(nt.reshape(nbh, _NCHUNK, nc), (1, 0, 2)).reshape(-1)
    dre_t = jnp.transpose(jnp.pad(dR_expanded, ((0, pad), (0, 0), (0, 0))),
                          (2, 1, 0))
    dr_t = jnp.pad(dR.T, ((0, 0), (0, pad)))
    pm_t = jnp.pad(pairwise_mask.T, ((0, 0), (0, pad)))
    weights = (fW1, fb1.reshape(1, f), fW2, fb2.reshape(1, f),
               f2out_W, f2out_b.reshape(1, d), out_W, out_b.reshape(1, d))

    outs = []
    for c in range(_NCHUNK):
        gat = _sc_gather(y, idx, c * ec, ec)
        outs.append(_fused(dre_t, dr_t, pm_t, gat.reshape(nbh, nc, f),
                           weights, c * nc, nc, nbh, d, f, g))
    return jnp.concatenate(outs, axis=0)[:n]


# speedup vs baseline: 1.0322x; 1.0322x over previous
import functools
import math

import jax
import jax.numpy as jnp
from jax import lax
from jax.experimental import pallas as pl
from jax.experimental.pallas import tpu as pltpu
from jax.experimental.pallas import tpu_sc as plsc


def _ssp(v):
    return jnp.maximum(v, 0.0) + jnp.log1p(jnp.exp(-jnp.abs(v))) - math.log(2.0)


def _in2f_body(x_ref, w_ref, o_ref):
    o_ref[...] = jnp.dot(x_ref[...], w_ref[...],
                         preferred_element_type=jnp.float32)


def _in2f(x, w):
    n, d = x.shape
    f = w.shape[1]
    blk = 2000
    return pl.pallas_call(
        _in2f_body,
        grid=(n // blk,),
        in_specs=[
            pl.BlockSpec((blk, d), lambda i: (i, 0)),
            pl.BlockSpec((d, f), lambda i: (0, 0)),
        ],
        out_specs=pl.BlockSpec((blk, f), lambda i: (i, 0)),
        out_shape=jax.ShapeDtypeStruct((n, f), jnp.float32),
    )(x, w)


_NC, _NS = 2, 16
_NW = _NC * _NS
_CH = 128
_K = 3
_PH = _CH * _K


def _sc_gather(table, idx, eoff, ecnt):
    n, f = table.shape
    per_w = ecnt // _NW
    nphase = per_w // _PH
    tail = per_w - nphase * _PH
    assert tail % 8 == 0 and tail <= _CH
    mesh = plsc.VectorSubcoreMesh(core_axis_name="c", subcore_axis_name="s")

    @functools.partial(
        pl.kernel,
        out_type=jax.ShapeDtypeStruct((ecnt, f), jnp.float32),
        mesh=mesh,
        scratch_types=[
            pltpu.VMEM((per_w,), jnp.int32),
            pltpu.VMEM((_PH, f), jnp.float32),
            pltpu.VMEM((_PH, f), jnp.float32),
            pltpu.SemaphoreType.DMA,
            pltpu.SemaphoreType.DMA,
        ],
    )
    def k(table_hbm, idx_hbm, out_hbm, idx_v, buf_a, buf_b, sem_a, sem_b):
        wid = lax.axis_index("s") * _NC + lax.axis_index("c")
        base = wid * per_w
        pltpu.sync_copy(idx_hbm.at[pl.ds(eoff + base, per_w)], idx_v)

        def fire(buf, sem, p):
            for j in range(_K):
                pltpu.async_copy(
                    table_hbm.at[idx_v.at[pl.ds(p * _PH + j * _CH, _CH)]],
                    buf.at[pl.ds(j * _CH, _CH)], sem)

        def drain(buf, sem):
            pltpu.make_async_copy(
                table_hbm.at[pl.ds(0, _PH)], buf, sem).wait()

        def flush(buf, p):
            pltpu.sync_copy(buf, out_hbm.at[pl.ds(base + p * _PH, _PH)])

        last = nphase - 1
        fire(buf_a, sem_a, 0)

        def body(i, carry):
            p = 2 * i
            fire(buf_b, sem_b, p + 1)
            drain(buf_a, sem_a)
            flush(buf_a, p)
            fire(buf_a, sem_a, jnp.minimum(p + 2, last))
            drain(buf_b, sem_b)
            flush(buf_b, p + 1)
            return carry

        lax.fori_loop(0, nphase // 2, body, 0, unroll=False)
        drain(buf_a, sem_a)
        if nphase % 2:
            flush(buf_a, last)
        if tail:
            pltpu.async_copy(
                table_hbm.at[idx_v.at[pl.ds(nphase * _PH, tail)]],
                buf_a.at[pl.ds(0, tail)], sem_a).wait()
            pltpu.sync_copy(buf_a.at[pl.ds(0, tail)],
                            out_hbm.at[pl.ds(base + nphase * _PH, tail)])

    return k(table, idx)


def _fused_body(dre_ref, dr_ref, pm_ref, gat_ref,
                fw1_ref, fb1_ref, fw2_ref, fb2_ref,
                f2o_ref, f2ob_ref, ow_ref, ob_ref, o_ref,
                *, blk_n, nbh):
    h3 = lax.dot_general(dre_ref[...], fw1_ref[...], (((0,), (0,)), ((), ())),
                         preferred_element_type=jnp.float32)
    h3 = _ssp(h3 + fb1_ref[...])
    w3 = lax.dot_general(h3, fw2_ref[...], (((2,), (0,)), ((), ())),
                         preferred_element_type=jnp.float32) + fb2_ref[...]
    dr = dr_ref[...]
    cut_t = 0.5 * (jnp.cos(dr * (math.pi / 5.0)) + 1.0)
    cut_t = cut_t * (dr < 5.0).astype(jnp.float32) * pm_ref[...]
    z3 = gat_ref[...] * w3 * cut_t[:, :, None]
    za = z3.sum(axis=0)
    ya = _ssp(jnp.dot(za, f2o_ref[...],
                      preferred_element_type=jnp.float32) + f2ob_ref[...])
    o_ref[...] = jnp.dot(ya, ow_ref[...],
                         preferred_element_type=jnp.float32) + ob_ref[...]


def _fused(dre_t, dr_t, pm_t, gat3, weights, noff, ncnt, nbh, d, f, g):
    fw1, fb1, fw2, fb2, f2o, f2ob, ow, ob = weights
    blk_n = 512
    b0 = noff // blk_n
    grid = (ncnt // blk_n,)
    body = functools.partial(_fused_body, blk_n=blk_n, nbh=nbh)
    return pl.pallas_call(
        body,
        grid=grid,
        in_specs=[
            pl.BlockSpec((g, nbh, blk_n), lambda i: (0, 0, i + b0)),
            pl.BlockSpec((nbh, blk_n), lambda i: (0, i + b0)),
            pl.BlockSpec((nbh, blk_n), lambda i: (0, i + b0)),
            pl.BlockSpec((nbh, blk_n, f), lambda i: (0, i, 0)),
            pl.BlockSpec((g, f), lambda i: (0, 0)),
            pl.BlockSpec((1, f), lambda i: (0, 0)),
            pl.BlockSpec((f, f), lambda i: (0, 0)),
            pl.BlockSpec((1, f), lambda i: (0, 0)),
            pl.BlockSpec((f, d), lambda i: (0, 0)),
            pl.BlockSpec((1, d), lambda i: (0, 0)),
            pl.BlockSpec((d, d), lambda i: (0, 0)),
            pl.BlockSpec((1, d), lambda i: (0, 0)),
        ],
        out_specs=pl.BlockSpec((blk_n, d), lambda i: (i, 0)),
        out_shape=jax.ShapeDtypeStruct((ncnt, d), jnp.float32),
    )(dre_t, dr_t, pm_t, gat3, fw1, fb1, fw2, fb2, f2o, f2ob, ow, ob)


_NCHUNK = 5


def kernel(x, dR, neighbors, pairwise_mask, dR_expanded,
           fW1, fb1, fW2, fb2, in2f_W, f2out_W, f2out_b, out_W, out_b):
    n, nbh = neighbors.shape
    d = x.shape[1]
    f = in2f_W.shape[1]
    g = dR_expanded.shape[2]

    y = _in2f(x, in2f_W)
    np_ = 10240 if n == 10000 else ((n + 639) // 640) * 640
    pad = np_ - n
    nc = np_ // _NCHUNK
    ec = nc * nbh
    pad_idx = jnp.broadcast_to(((jnp.arange(pad) * 977) % n)[None, :],
                               (nbh, pad)).astype(jnp.int32)
    nt = jnp.concatenate([neighbors.T.astype(jnp.int32), pad_idx], axis=1)
    idx = jnp.transpose(nt.reshape(nbh, _NCHUNK, nc), (1, 0, 2)).reshape(-1)
    dre_t = jnp.transpose(jnp.pad(dR_expanded, ((0, pad), (0, 0), (0, 0))),
                          (2, 1, 0))
    dr_t = jnp.pad(dR.T, ((0, 0), (0, pad)))
    pm_t = jnp.pad(pairwise_mask.T, ((0, 0), (0, pad)))
    weights = (fW1, fb1.reshape(1, f), fW2, fb2.reshape(1, f),
               f2out_W, f2out_b.reshape(1, d), out_W, out_b.reshape(1, d))

    outs = []
    for c in range(_NCHUNK):
        gat = _sc_gather(y, idx, c * ec, ec)
        outs.append(_fused(dre_t, dr_t, pm_t, gat.reshape(nbh, nc, f),
                           weights, c * nc, nc, nbh, d, f, g))
    return jnp.concatenate(outs, axis=0)[:n]
